# baseline (speedup 1.0000x reference)
import functools

import jax
import jax.numpy as jnp
from jax.experimental import pallas as pl
from jax.experimental.pallas import tpu as pltpu


def _fused_pass_kernel(x_ref, w_ref, a_ref, o_ref, z_ref, *, out_bf16):
    @pl.when(pl.program_id(0) == 0)
    def _():
        z_ref[...] = jnp.dot(
            x_ref[...].astype(jnp.bfloat16),
            w_ref[...].astype(jnp.bfloat16),
            preferred_element_type=jnp.float32,
        ).astype(jnp.bfloat16)

    acc = jnp.dot(a_ref[...].astype(jnp.bfloat16), z_ref[...],
                  preferred_element_type=jnp.float32)
    o_ref[...] = acc.astype(jnp.bfloat16) if out_bf16 else acc


def _fused_pass(a, x, w, bm, out_bf16):
    n = a.shape[0]
    d = w.shape[1]
    out_dtype = jnp.bfloat16 if out_bf16 else jnp.float32
    return pl.pallas_call(
        functools.partial(_fused_pass_kernel, out_bf16=out_bf16),
        grid=(n // bm,),
        in_specs=[
            pl.BlockSpec((x.shape[0], d), lambda i: (0, 0)),
            pl.BlockSpec((d, d), lambda i: (0, 0)),
            pl.BlockSpec((bm, n), lambda i: (i, 0)),
        ],
        out_specs=pl.BlockSpec((bm, d), lambda i: (i, 0)),
        out_shape=jax.ShapeDtypeStruct((n, d), out_dtype),
        scratch_shapes=[pltpu.VMEM((n, d), jnp.bfloat16)],
    )(x, w, a)


def kernel(X, A_hat, W1, W2):
    n = A_hat.shape[0]
    bm = 200 if n % 200 == 0 else n
    h = _fused_pass(A_hat, X, W1, bm, out_bf16=True)
    y = _fused_pass(A_hat, h, W2, bm, out_bf16=False)
    return y


# speedup vs baseline: 1.0451x; 1.0451x over previous
import functools

import jax
import jax.numpy as jnp
from jax.experimental import pallas as pl
from jax.experimental.pallas import tpu as pltpu


def _gcn_kernel(x_ref, w1_ref, w2_ref, a_ref, o_ref, z_ref, h_ref, *, bm):
    p = pl.program_id(0)
    i = pl.program_id(1)

    @pl.when((p == 0) & (i == 0))
    def _():
        z_ref[...] = jnp.dot(
            x_ref[...].astype(jnp.bfloat16),
            w1_ref[...].astype(jnp.bfloat16),
            preferred_element_type=jnp.float32,
        ).astype(jnp.bfloat16)

    @pl.when((p == 1) & (i == 0))
    def _():
        z_ref[...] = jnp.dot(
            h_ref[...],
            w2_ref[...].astype(jnp.bfloat16),
            preferred_element_type=jnp.float32,
        ).astype(jnp.bfloat16)

    acc = jnp.dot(a_ref[...].astype(jnp.bfloat16), z_ref[...],
                  preferred_element_type=jnp.float32)

    @pl.when(p == 0)
    def _():
        nblk = pl.num_programs(1)
        row = (nblk - 1 - i) * bm
        h_ref[pl.ds(row, bm), :] = acc.astype(jnp.bfloat16)

    @pl.when(p == 1)
    def _():
        o_ref[...] = acc


def kernel(X, A_hat, W1, W2):
    n = A_hat.shape[0]
    d = W1.shape[1]
    bm = 400 if n % 400 == 0 else n
    nblk = n // bm

    def a_map(p, i):
        return ((1 - p) * (nblk - 1) + (2 * p - 1) * i, 0)

    return pl.pallas_call(
        functools.partial(_gcn_kernel, bm=bm),
        grid=(2, nblk),
        in_specs=[
            pl.BlockSpec((n, d), lambda p, i: (0, 0)),
            pl.BlockSpec((d, d), lambda p, i: (0, 0)),
            pl.BlockSpec((d, d), lambda p, i: (0, 0)),
            pl.BlockSpec((bm, n), a_map),
        ],
        out_specs=pl.BlockSpec((bm, d), lambda p, i: (p * i, 0)),
        out_shape=jax.ShapeDtypeStruct((n, d), jnp.float32),
        scratch_shapes=[
            pltpu.VMEM((n, d), jnp.bfloat16),
            pltpu.VMEM((n, d), jnp.bfloat16),
        ],
    )(X, W1, W2, A_hat)
